# Initial kernel scaffold; baseline (speedup 1.0000x reference)
import jax
import jax.numpy as jnp
from jax.experimental import pallas as pl


def kernel(boxes, scores):
    raise NotImplementedError("write your pallas kernel here")


# speedup vs baseline: 71.7131x; 71.7131x over previous
import functools

import jax
import jax.numpy as jnp
from jax import lax
from jax.experimental import pallas as pl
from jax.experimental.pallas import tpu as pltpu

N = 5000
NP = 5120
BLK = 128
NBLK = NP // BLK
IOU_THRESH = 0.2
SCORE_THRESH = 0.2


def _nms_block_kernel(b_ref, bT_ref, s_ref, out_ref, mask_ref):
    i = pl.program_id(0)

    @pl.when(i == 0)
    def _init():
        mask_ref[...] = jnp.ones((1, NP), jnp.float32)

    x1c = bT_ref[0:1, :]
    y1c = bT_ref[1:2, :]
    x2c = bT_ref[2:3, :]
    y2c = bT_ref[3:4, :]
    areas_c = jnp.maximum(x2c - x1c, 0.0) * jnp.maximum(y2c - y1c, 0.0)

    x1r = b_ref[:, 0:1]
    y1r = b_ref[:, 1:2]
    x2r = b_ref[:, 2:3]
    y2r = b_ref[:, 3:4]
    areas_r = jnp.maximum(x2r - x1r, 0.0) * jnp.maximum(y2r - y1r, 0.0)

    xx1 = jnp.maximum(x1r, x1c)
    yy1 = jnp.maximum(y1r, y1c)
    xx2 = jnp.minimum(x2r, x2c)
    yy2 = jnp.minimum(y2r, y2c)
    w = jnp.maximum(xx2 - xx1, 0.0)
    h = jnp.maximum(yy2 - yy1, 0.0)
    inter = w * h
    union = areas_r + areas_c - inter
    iou = inter / (union + 1e-9)

    colidx = lax.broadcasted_iota(jnp.int32, (BLK, NP), 1)
    rowidx = lax.broadcasted_iota(jnp.int32, (BLK, NP), 0) + i * BLK
    s_bits = jnp.where((iou > IOU_THRESH) & (colidx > rowidx), 1.0, 0.0)

    base = pl.multiple_of(i * BLK, BLK)
    x1cb = bT_ref[0:1, pl.ds(base, BLK)]
    y1cb = bT_ref[1:2, pl.ds(base, BLK)]
    x2cb = bT_ref[2:3, pl.ds(base, BLK)]
    y2cb = bT_ref[3:4, pl.ds(base, BLK)]
    areas_cb = jnp.maximum(x2cb - x1cb, 0.0) * jnp.maximum(y2cb - y1cb, 0.0)
    wb = jnp.maximum(jnp.minimum(x2r, x2cb) - jnp.maximum(x1r, x1cb), 0.0)
    hb = jnp.maximum(jnp.minimum(y2r, y2cb) - jnp.maximum(y1r, y1cb), 0.0)
    inter_b = wb * hb
    iou_b = inter_b / (areas_r + areas_cb - inter_b + 1e-9)
    tri = (lax.broadcasted_iota(jnp.int32, (BLK, BLK), 1)
           > lax.broadcasted_iota(jnp.int32, (BLK, BLK), 0))
    s_local = jnp.where((iou_b > IOU_THRESH) & tri, 1.0, 0.0)
    m0 = mask_ref[0:1, pl.ds(base, BLK)]

    def cond(carry):
        return carry[1]

    def body(carry):
        m, _ = carry
        sup = jax.lax.dot_general(
            m, s_local, (((1,), (0,)), ((), ())),
            preferred_element_type=jnp.float32)
        m_new = jnp.where(sup > 0.0, 0.0, m0)
        return m_new, jnp.any(m_new != m)

    m_fix, _ = lax.while_loop(cond, body, (m0, jnp.bool_(True)))
    mask_ref[0:1, pl.ds(base, BLK)] = m_fix

    sup_tail = jax.lax.dot_general(
        m_fix, s_bits, (((1,), (0,)), ((), ())),
        preferred_element_type=jnp.float32)
    mask_full = jnp.where(sup_tail > 0.0, 0.0, mask_ref[...])
    mask_ref[...] = mask_full

    m_rows = jnp.reshape(m_fix, (BLK, 1))
    keep = m_rows * jnp.where(s_ref[...] > SCORE_THRESH, 1.0, 0.0)
    out_ref[...] = jnp.concatenate([b_ref[...], s_ref[...]], axis=1) * keep


@jax.jit
def kernel(boxes, scores):
    order = jnp.argsort(-scores)
    b = jnp.take(boxes, order, axis=0)
    s = jnp.take(scores, order, axis=0)
    bp = jnp.zeros((NP, 4), jnp.float32).at[:N].set(b)
    sp = jnp.zeros((NP, 1), jnp.float32).at[:N, 0].set(s)
    bT = bp.T

    out = pl.pallas_call(
        _nms_block_kernel,
        grid=(NBLK,),
        in_specs=[
            pl.BlockSpec((BLK, 4), lambda i: (i, 0)),
            pl.BlockSpec((4, NP), lambda i: (0, 0)),
            pl.BlockSpec((BLK, 1), lambda i: (i, 0)),
        ],
        out_specs=pl.BlockSpec((BLK, 5), lambda i: (i, 0)),
        out_shape=jax.ShapeDtypeStruct((NP, 5), jnp.float32),
        scratch_shapes=[pltpu.VMEM((1, NP), jnp.float32)],
    )(bp, bT, sp)
    return out[:N]
